# baseline (speedup 1.0000x reference)
import functools

import jax
import jax.numpy as jnp
from jax import lax
from jax.experimental import pallas as pl
from jax.experimental.pallas import tpu as pltpu
from jax.experimental.pallas import tpu_sc as plsc

NC = 2
NS = 16
NW = NC * NS


def _gather_pool_pass(tbl, idx_ref, idx_base, G, CBs, nch, rb0, rb1,
                      out_ref, sem0, sem1, out_base):
    nrows = CBs * G

    def fire(c, rb, sem):
        off = idx_base + c * nrows
        pltpu.async_copy(tbl.at[idx_ref.at[pl.ds(off, nrows)]],
                         rb.at[pl.ds(0, nrows)], sem)

    def drain(rb, sem):
        pltpu.make_async_copy(tbl.at[pl.ds(0, nrows)],
                              rb.at[pl.ds(0, nrows)], sem).wait()

    def compute(c, rb):
        def b_body(bl, _):
            row = out_base + c * CBs + bl
            base = bl * G
            accs = [[rb[base + p, pl.ds(dc * 16, 16)] for p in range(2)]
                    for dc in range(4)]
            for k in range(2, G):
                p = k % 2
                for dc in range(4):
                    accs[dc][p] = accs[dc][p] + rb[base + k, pl.ds(dc * 16, 16)]
            for dc in range(4):
                out_ref[row, pl.ds(dc * 16, 16)] = accs[dc][0] + accs[dc][1]
            return _
        lax.fori_loop(0, CBs, b_body, None)

    fire(0, rb0, sem0)

    def body2(j, _):
        c0 = 2 * j
        drain(rb0, sem0)
        fire(c0 + 1, rb1, sem1)
        compute(c0, rb0)
        drain(rb1, sem1)
        fire(jnp.minimum(c0 + 2, nch - 1), rb0, sem0)
        compute(c0 + 1, rb1)
        return _
    lax.fori_loop(0, nch // 2, body2, None)
    drain(rb0, sem0)


def _sc_body(B, L, K, D, u_hbm, v_hbm, up_hbm, vp_hbm, vn_hbm,
             su_hbm, sv_hbm, sn_hbm, iu, iv, inb, rb0, rb1,
             ou, ov, on, sem0, sem1, semi):
    bpw = B // NW
    wid = lax.axis_index("s") * NC + lax.axis_index("c")

    pltpu.async_copy(up_hbm.at[pl.ds(wid * (bpw * L), bpw * L)], iu, semi)
    pltpu.async_copy(vp_hbm.at[pl.ds(wid * (bpw * L), bpw * L)], iv, semi)
    pltpu.async_copy(vn_hbm.at[pl.ds(wid * (bpw * K), bpw * K)], inb, semi)
    pltpu.make_async_copy(up_hbm.at[pl.ds(0, bpw * L)], iu, semi).wait()
    pltpu.make_async_copy(vp_hbm.at[pl.ds(0, bpw * L)], iv, semi).wait()
    pltpu.make_async_copy(vn_hbm.at[pl.ds(0, bpw * K)], inb, semi).wait()

    CB_UV = 16
    CB_N = 4
    _gather_pool_pass(u_hbm, iu, 0, L, CB_UV, bpw // CB_UV, rb0, rb1,
                      ou, sem0, sem1, 0)
    _gather_pool_pass(v_hbm, iv, 0, L, CB_UV, bpw // CB_UV, rb0, rb1,
                      ov, sem0, sem1, 0)
    _gather_pool_pass(v_hbm, inb, 0, K, CB_N, bpw // CB_N, rb0, rb1,
                      on, sem0, sem1, 0)

    base = wid * bpw
    pltpu.sync_copy(ou, su_hbm.at[pl.ds(base, bpw)])
    pltpu.sync_copy(ov, sv_hbm.at[pl.ds(base, bpw)])
    pltpu.sync_copy(on, sn_hbm.at[pl.ds(base, bpw)])


def _tc_body(scale, su_ref, sv_ref, sn_ref, o_ref):
    su = su_ref[...]
    sv = sv_ref[...]
    sn = sn_ref[...]
    s = jnp.sum(su * sv, axis=1) * scale
    ns = jnp.sum(su * sn, axis=1) * scale

    def logsig(x):
        return jnp.minimum(x, 0.0) - jnp.log1p(jnp.exp(-jnp.abs(x)))

    loss = logsig(s) + logsig(-ns)
    o_ref[...] = jnp.reshape(-jnp.sum(loss), (1, 1))


def kernel(u_emb, v_emb, u_pos, v_pos, v_neg, batch_size):
    B, L = u_pos.shape
    NNEG = v_neg.shape[2]
    D = u_emb.shape[1]
    K = L * NNEG
    bpw = B // NW

    up = u_pos.astype(jnp.int32).reshape(-1)
    vp = v_pos.astype(jnp.int32).reshape(-1)
    vn = v_neg.astype(jnp.int32).reshape(-1)

    sc = functools.partial(
        pl.kernel,
        mesh=plsc.VectorSubcoreMesh(core_axis_name="c", subcore_axis_name="s"),
        out_type=[jax.ShapeDtypeStruct((B, D), jnp.float32)] * 3,
        scratch_types=[
            pltpu.VMEM((bpw * L,), jnp.int32),
            pltpu.VMEM((bpw * L,), jnp.int32),
            pltpu.VMEM((bpw * K,), jnp.int32),
            pltpu.VMEM((400, D), jnp.float32),
            pltpu.VMEM((400, D), jnp.float32),
            pltpu.VMEM((bpw, D), jnp.float32),
            pltpu.VMEM((bpw, D), jnp.float32),
            pltpu.VMEM((bpw, D), jnp.float32),
            pltpu.SemaphoreType.DMA,
            pltpu.SemaphoreType.DMA,
            pltpu.SemaphoreType.DMA,
        ],
        compiler_params=pltpu.CompilerParams(use_tc_tiling_on_sc=False),
    )(functools.partial(_sc_body, B, L, K, D))
    su, sv, sn = sc(u_emb, v_emb, up, vp, vn)

    out = pl.pallas_call(
        functools.partial(_tc_body, 1.0 / float(L * L)),
        out_shape=jax.ShapeDtypeStruct((1, 1), jnp.float32),
    )(su, sv, sn)
    return out[0, 0] / jnp.asarray(batch_size, jnp.float32)


# speedup vs baseline: 1.7367x; 1.7367x over previous
import functools

import jax
import jax.numpy as jnp
from jax import lax
from jax.experimental import pallas as pl
from jax.experimental.pallas import tpu as pltpu
from jax.experimental.pallas import tpu_sc as plsc

NC = 2
NS = 16
NW = NC * NS


def _gather_pool_pass(tbl, idx_ref, idx_base, G, CBs, nch, rb0, rb1,
                      out_ref, sem0, sem1, out_base):
    nrows = CBs * G

    def fire(c, rb, sem):
        off = idx_base + c * nrows
        pltpu.async_copy(tbl.at[idx_ref.at[pl.ds(off, nrows)]],
                         rb.at[pl.ds(0, nrows)], sem)

    def drain(rb, sem):
        pltpu.make_async_copy(tbl.at[pl.ds(0, nrows)],
                              rb.at[pl.ds(0, nrows)], sem).wait()

    def compute(c, rb):
        def b_body(bl, _):
            row = out_base + c * CBs + bl
            base = bl * G
            accs = [[rb[base + p, pl.ds(dc * 16, 16)] for p in range(2)]
                    for dc in range(4)]
            for k in range(2, G):
                p = k % 2
                for dc in range(4):
                    accs[dc][p] = accs[dc][p] + rb[base + k, pl.ds(dc * 16, 16)]
            for dc in range(4):
                out_ref[row, pl.ds(dc * 16, 16)] = accs[dc][0] + accs[dc][1]
            return _
        lax.fori_loop(0, CBs, b_body, None)

    fire(0, rb0, sem0)

    def body2(j, _):
        c0 = 2 * j
        drain(rb0, sem0)
        fire(c0 + 1, rb1, sem1)
        compute(c0, rb0)
        drain(rb1, sem1)
        fire(jnp.minimum(c0 + 2, nch - 1), rb0, sem0)
        compute(c0 + 1, rb1)
        return _
    lax.fori_loop(0, nch // 2, body2, None)
    drain(rb0, sem0)


def _sc_body(B, L, K, D, u_hbm, v_hbm, up_hbm, vp_hbm, vn_hbm,
             su_hbm, sv_hbm, sn_hbm, iu, iv, inb, rb0, rb1,
             ou, ov, on, sem0, sem1, semi):
    bpw = B // NW
    wid = lax.axis_index("s") * NC + lax.axis_index("c")

    pltpu.async_copy(up_hbm.at[pl.ds(wid * (bpw * L), bpw * L)], iu, semi)
    pltpu.async_copy(vp_hbm.at[pl.ds(wid * (bpw * L), bpw * L)], iv, semi)
    pltpu.async_copy(vn_hbm.at[pl.ds(wid * (bpw * K), bpw * K)], inb, semi)
    pltpu.make_async_copy(up_hbm.at[pl.ds(0, bpw * L)], iu, semi).wait()
    pltpu.make_async_copy(vp_hbm.at[pl.ds(0, bpw * L)], iv, semi).wait()
    pltpu.make_async_copy(vn_hbm.at[pl.ds(0, bpw * K)], inb, semi).wait()

    CB_UV = 16
    CB_N = 4
    _gather_pool_pass(u_hbm, iu, 0, L, CB_UV, bpw // CB_UV, rb0, rb1,
                      ou, sem0, sem1, 0)
    _gather_pool_pass(v_hbm, iv, 0, L, CB_UV, bpw // CB_UV, rb0, rb1,
                      ov, sem0, sem1, 0)
    _gather_pool_pass(v_hbm, inb, 0, K, CB_N, bpw // CB_N, rb0, rb1,
                      on, sem0, sem1, 0)

    base = wid * bpw
    pltpu.sync_copy(ou, su_hbm.at[pl.ds(base, bpw)])
    pltpu.sync_copy(ov, sv_hbm.at[pl.ds(base, bpw)])
    pltpu.sync_copy(on, sn_hbm.at[pl.ds(base, bpw)])


VB = 8192


def _repack_body(xt_ref, eye_ref, o_ref):
    x = xt_ref[...]
    y = jax.lax.dot_general(x, eye_ref[...], (((0,), (0,)), ((), ())),
                            preferred_element_type=jnp.float32)
    h = y.shape[0] // 2
    o_ref[:, 0:64] = y[0:h, :]
    o_ref[:, 64:128] = y[h:, :]


def _repack(t, D):
    V = t.shape[0]
    grid = (V + VB - 1) // VB
    eye = jnp.eye(D, dtype=jnp.float32)
    return pl.pallas_call(
        _repack_body,
        grid=(grid,),
        in_specs=[
            pl.BlockSpec((D, VB), lambda i: (0, i)),
            pl.BlockSpec((D, D), lambda i: (0, 0)),
        ],
        out_specs=pl.BlockSpec((VB // 2, 2 * D), lambda i: (i, 0)),
        out_shape=jax.ShapeDtypeStruct((grid * (VB // 2), 2 * D), jnp.float32),
    )(t.T, eye)


def _remap_body(up_ref, vp_ref, vn_ref, ou_ref, ov_ref, on_ref):
    def remap(v):
        return ((v >> 13) << 13) | ((v & 4095) << 1) | ((v >> 12) & 1)
    ou_ref[...] = remap(up_ref[...])
    ov_ref[...] = remap(vp_ref[...])
    on_ref[...] = remap(vn_ref[...])


def _remap_indices(up, vp, vn):
    n_p, n_n = up.shape[0], vn.shape[0]
    ou, ov, on = pl.pallas_call(
        _remap_body,
        out_shape=[jax.ShapeDtypeStruct((n_p // 128, 128), jnp.int32)] * 2
        + [jax.ShapeDtypeStruct((n_n // 128, 128), jnp.int32)],
    )(up.reshape(n_p // 128, 128), vp.reshape(n_p // 128, 128),
      vn.reshape(n_n // 128, 128))
    return ou.reshape(-1), ov.reshape(-1), on.reshape(-1)


def _tc_body(scale, su_ref, sv_ref, sn_ref, o_ref):
    su = su_ref[...]
    sv = sv_ref[...]
    sn = sn_ref[...]
    s = jnp.sum(su * sv, axis=1) * scale
    ns = jnp.sum(su * sn, axis=1) * scale

    def logsig(x):
        return jnp.minimum(x, 0.0) - jnp.log1p(jnp.exp(-jnp.abs(x)))

    loss = logsig(s) + logsig(-ns)
    o_ref[...] = jnp.reshape(-jnp.sum(loss), (1, 1))


def kernel(u_emb, v_emb, u_pos, v_pos, v_neg, batch_size):
    B, L = u_pos.shape
    NNEG = v_neg.shape[2]
    D = u_emb.shape[1]
    K = L * NNEG
    bpw = B // NW

    up = u_pos.astype(jnp.int32).reshape(-1)
    vp = v_pos.astype(jnp.int32).reshape(-1)
    vn = v_neg.astype(jnp.int32).reshape(-1)
    up, vp, vn = _remap_indices(up, vp, vn)

    u_tbl = _repack(u_emb, D)
    v_tbl = _repack(v_emb, D)
    u_tbl = u_tbl.reshape(2 * u_tbl.shape[0], D)
    v_tbl = v_tbl.reshape(2 * v_tbl.shape[0], D)

    sc = functools.partial(
        pl.kernel,
        mesh=plsc.VectorSubcoreMesh(core_axis_name="c", subcore_axis_name="s"),
        out_type=[jax.ShapeDtypeStruct((B, D), jnp.float32)] * 3,
        scratch_types=[
            pltpu.VMEM((bpw * L,), jnp.int32),
            pltpu.VMEM((bpw * L,), jnp.int32),
            pltpu.VMEM((bpw * K,), jnp.int32),
            pltpu.VMEM((400, D), jnp.float32),
            pltpu.VMEM((400, D), jnp.float32),
            pltpu.VMEM((bpw, D), jnp.float32),
            pltpu.VMEM((bpw, D), jnp.float32),
            pltpu.VMEM((bpw, D), jnp.float32),
            pltpu.SemaphoreType.DMA,
            pltpu.SemaphoreType.DMA,
            pltpu.SemaphoreType.DMA,
        ],
        compiler_params=pltpu.CompilerParams(use_tc_tiling_on_sc=False),
    )(functools.partial(_sc_body, B, L, K, D))
    su, sv, sn = sc(u_tbl, v_tbl, up, vp, vn)

    out = pl.pallas_call(
        functools.partial(_tc_body, 1.0 / float(L * L)),
        out_shape=jax.ShapeDtypeStruct((1, 1), jnp.float32),
    )(su, sv, sn)
    return out[0, 0] / jnp.asarray(batch_size, jnp.float32)


# speedup vs baseline: 1.9055x; 1.0972x over previous
import functools

import jax
import jax.numpy as jnp
from jax import lax
from jax.experimental import pallas as pl
from jax.experimental.pallas import tpu as pltpu
from jax.experimental.pallas import tpu_sc as plsc

NC = 2
NS = 16
NW = NC * NS


def _gather_pool_pass(tbl, idx_ref, idx_base, G, CBs, nch, rb0, rb1,
                      out_ref, sem0, sem1, out_base):
    nrows = CBs * G

    def fire(c, rb, sem):
        off = idx_base + c * nrows
        pltpu.async_copy(tbl.at[idx_ref.at[pl.ds(off, nrows)]],
                         rb.at[pl.ds(0, nrows)], sem)

    def drain(rb, sem):
        pltpu.make_async_copy(tbl.at[pl.ds(0, nrows)],
                              rb.at[pl.ds(0, nrows)], sem).wait()

    def compute(c, rb):
        def b_body(bl, _):
            row = out_base + c * CBs + bl
            base = bl * G
            accs = [[rb[base + p, pl.ds(dc * 16, 16)] for p in range(2)]
                    for dc in range(4)]
            for k in range(2, G):
                p = k % 2
                for dc in range(4):
                    accs[dc][p] = accs[dc][p] + rb[base + k, pl.ds(dc * 16, 16)]
            for dc in range(4):
                out_ref[row, pl.ds(dc * 16, 16)] = accs[dc][0] + accs[dc][1]
            return _
        lax.fori_loop(0, CBs, b_body, None)

    fire(0, rb0, sem0)

    def body2(j, _):
        c0 = 2 * j
        drain(rb0, sem0)
        fire(c0 + 1, rb1, sem1)
        compute(c0, rb0)
        drain(rb1, sem1)
        fire(jnp.minimum(c0 + 2, nch - 1), rb0, sem0)
        compute(c0 + 1, rb1)
        return _
    lax.fori_loop(0, nch // 2, body2, None)
    drain(rb0, sem0)


def _sc_body(B, L, K, D, u_hbm, v_hbm, up_hbm, vp_hbm, vn_hbm,
             su_hbm, sv_hbm, sn_hbm, iu, iv, inb, rb0, rb1,
             ou, ov, on, sem0, sem1, semi):
    bpw = B // NW
    wid = lax.axis_index("s") * NC + lax.axis_index("c")

    pltpu.async_copy(up_hbm.at[pl.ds(wid * (bpw * L), bpw * L)], iu, semi)
    pltpu.async_copy(vp_hbm.at[pl.ds(wid * (bpw * L), bpw * L)], iv, semi)
    pltpu.async_copy(vn_hbm.at[pl.ds(wid * (bpw * K), bpw * K)], inb, semi)
    pltpu.make_async_copy(up_hbm.at[pl.ds(0, bpw * L)], iu, semi).wait()
    pltpu.make_async_copy(vp_hbm.at[pl.ds(0, bpw * L)], iv, semi).wait()
    pltpu.make_async_copy(vn_hbm.at[pl.ds(0, bpw * K)], inb, semi).wait()

    CB_UV = 16
    CB_N = 4
    _gather_pool_pass(u_hbm, iu, 0, L, CB_UV, bpw // CB_UV, rb0, rb1,
                      ou, sem0, sem1, 0)
    _gather_pool_pass(v_hbm, iv, 0, L, CB_UV, bpw // CB_UV, rb0, rb1,
                      ov, sem0, sem1, 0)
    _gather_pool_pass(v_hbm, inb, 0, K, CB_N, bpw // CB_N, rb0, rb1,
                      on, sem0, sem1, 0)

    base = wid * bpw
    pltpu.sync_copy(ou, su_hbm.at[pl.ds(base, bpw)])
    pltpu.sync_copy(ov, sv_hbm.at[pl.ds(base, bpw)])
    pltpu.sync_copy(on, sn_hbm.at[pl.ds(base, bpw)])


VB = 16384


def _repack_body(xt_ref, eye_ref, o_ref):
    x = xt_ref[...]
    y = jax.lax.dot_general(x, eye_ref[...], (((0,), (0,)), ((), ())),
                            preferred_element_type=jnp.float32)
    h = y.shape[0] // 2
    o_ref[:, 0:64] = y[0:h, :]
    o_ref[:, 64:128] = y[h:, :]


def _repack(t, D):
    V = t.shape[0]
    grid = (V + VB - 1) // VB
    eye = jnp.eye(D, dtype=jnp.float32)
    return pl.pallas_call(
        _repack_body,
        grid=(grid,),
        in_specs=[
            pl.BlockSpec((D, VB), lambda i: (0, i)),
            pl.BlockSpec((D, D), lambda i: (0, 0)),
        ],
        out_specs=pl.BlockSpec((VB // 2, 2 * D), lambda i: (i, 0)),
        out_shape=jax.ShapeDtypeStruct((grid * (VB // 2), 2 * D), jnp.float32),
    )(t.T, eye)


def _remap_body(up_ref, vp_ref, vn_ref, ou_ref, ov_ref, on_ref):
    sh = VB.bit_length() - 1

    def remap(v):
        return ((v >> sh) << sh) | ((v & (VB // 2 - 1)) << 1) | (
            (v >> (sh - 1)) & 1)
    ou_ref[...] = remap(up_ref[...])
    ov_ref[...] = remap(vp_ref[...])
    on_ref[...] = remap(vn_ref[...])


def _remap_indices(up, vp, vn):
    n_p, n_n = up.shape[0], vn.shape[0]
    ou, ov, on = pl.pallas_call(
        _remap_body,
        out_shape=[jax.ShapeDtypeStruct((n_p // 128, 128), jnp.int32)] * 2
        + [jax.ShapeDtypeStruct((n_n // 128, 128), jnp.int32)],
    )(up.reshape(n_p // 128, 128), vp.reshape(n_p // 128, 128),
      vn.reshape(n_n // 128, 128))
    return ou.reshape(-1), ov.reshape(-1), on.reshape(-1)


def _tc_body(scale, su_ref, sv_ref, sn_ref, o_ref):
    su = su_ref[...]
    sv = sv_ref[...]
    sn = sn_ref[...]
    s = jnp.sum(su * sv, axis=1) * scale
    ns = jnp.sum(su * sn, axis=1) * scale

    def logsig(x):
        return jnp.minimum(x, 0.0) - jnp.log1p(jnp.exp(-jnp.abs(x)))

    loss = logsig(s) + logsig(-ns)
    o_ref[...] = jnp.reshape(-jnp.sum(loss), (1, 1))


def kernel(u_emb, v_emb, u_pos, v_pos, v_neg, batch_size):
    B, L = u_pos.shape
    NNEG = v_neg.shape[2]
    D = u_emb.shape[1]
    K = L * NNEG
    bpw = B // NW

    up = u_pos.astype(jnp.int32).reshape(-1)
    vp = v_pos.astype(jnp.int32).reshape(-1)
    vn = v_neg.astype(jnp.int32).reshape(-1)
    up, vp, vn = _remap_indices(up, vp, vn)

    u_tbl = _repack(u_emb, D)
    v_tbl = _repack(v_emb, D)
    u_tbl = u_tbl.reshape(2 * u_tbl.shape[0], D)
    v_tbl = v_tbl.reshape(2 * v_tbl.shape[0], D)

    sc = functools.partial(
        pl.kernel,
        mesh=plsc.VectorSubcoreMesh(core_axis_name="c", subcore_axis_name="s"),
        out_type=[jax.ShapeDtypeStruct((B, D), jnp.float32)] * 3,
        scratch_types=[
            pltpu.VMEM((bpw * L,), jnp.int32),
            pltpu.VMEM((bpw * L,), jnp.int32),
            pltpu.VMEM((bpw * K,), jnp.int32),
            pltpu.VMEM((400, D), jnp.float32),
            pltpu.VMEM((400, D), jnp.float32),
            pltpu.VMEM((bpw, D), jnp.float32),
            pltpu.VMEM((bpw, D), jnp.float32),
            pltpu.VMEM((bpw, D), jnp.float32),
            pltpu.SemaphoreType.DMA,
            pltpu.SemaphoreType.DMA,
            pltpu.SemaphoreType.DMA,
        ],
        compiler_params=pltpu.CompilerParams(use_tc_tiling_on_sc=False),
    )(functools.partial(_sc_body, B, L, K, D))
    su, sv, sn = sc(u_tbl, v_tbl, up, vp, vn)

    out = pl.pallas_call(
        functools.partial(_tc_body, 1.0 / float(L * L)),
        out_shape=jax.ShapeDtypeStruct((1, 1), jnp.float32),
    )(su, sv, sn)
    return out[0, 0] / jnp.asarray(batch_size, jnp.float32)


# speedup vs baseline: 1.9960x; 1.0475x over previous
import functools

import jax
import jax.numpy as jnp
from jax import lax
from jax.experimental import pallas as pl
from jax.experimental.pallas import tpu as pltpu
from jax.experimental.pallas import tpu_sc as plsc

NC = 2
NS = 16
NW = NC * NS


def _gather_pool_pass(tbl, idx_ref, idx_base, G, CBs, nch, rb0, rb1,
                      out_ref, sem0, sem1, out_base):
    nrows = CBs * G

    def fire(c, rb, sem):
        off = idx_base + c * nrows
        pltpu.async_copy(tbl.at[idx_ref.at[pl.ds(off, nrows)]],
                         rb.at[pl.ds(0, nrows)], sem)

    def drain(rb, sem):
        pltpu.make_async_copy(tbl.at[pl.ds(0, nrows)],
                              rb.at[pl.ds(0, nrows)], sem).wait()

    def compute(c, rb):
        def b_body(bl, _):
            row = out_base + c * CBs + bl
            base = bl * G
            accs = [[rb[base + p, pl.ds(dc * 16, 16)] for p in range(2)]
                    for dc in range(4)]
            for k in range(2, G):
                p = k % 2
                for dc in range(4):
                    accs[dc][p] = accs[dc][p] + rb[base + k, pl.ds(dc * 16, 16)]
            for dc in range(4):
                out_ref[row, pl.ds(dc * 16, 16)] = accs[dc][0] + accs[dc][1]
            return _
        lax.fori_loop(0, CBs, b_body, None)

    fire(0, rb0, sem0)

    def body2(j, _):
        c0 = 2 * j
        drain(rb0, sem0)
        fire(c0 + 1, rb1, sem1)
        compute(c0, rb0)
        drain(rb1, sem1)
        fire(jnp.minimum(c0 + 2, nch - 1), rb0, sem0)
        compute(c0 + 1, rb1)
        return _
    lax.fori_loop(0, nch // 2, body2, None)
    drain(rb0, sem0)


def _sc_body(B, L, K, D, u_hbm, v_hbm, up_hbm, vp_hbm, vn_hbm,
             su_hbm, sv_hbm, sn_hbm, iu, iv, inb, rb0, rb1,
             ou, ov, on, sem0, sem1, semi):
    bpw = B // NW
    wid = lax.axis_index("s") * NC + lax.axis_index("c")

    pltpu.async_copy(up_hbm.at[pl.ds(wid * (bpw * L), bpw * L)], iu, semi)
    pltpu.async_copy(vp_hbm.at[pl.ds(wid * (bpw * L), bpw * L)], iv, semi)
    pltpu.async_copy(vn_hbm.at[pl.ds(wid * (bpw * K), bpw * K)], inb, semi)
    pltpu.make_async_copy(up_hbm.at[pl.ds(0, bpw * L)], iu, semi).wait()
    pltpu.make_async_copy(vp_hbm.at[pl.ds(0, bpw * L)], iv, semi).wait()
    pltpu.make_async_copy(vn_hbm.at[pl.ds(0, bpw * K)], inb, semi).wait()

    CB_UV = 16
    CB_N = 4
    _gather_pool_pass(u_hbm, iu, 0, L, CB_UV, bpw // CB_UV, rb0, rb1,
                      ou, sem0, sem1, 0)
    _gather_pool_pass(v_hbm, iv, 0, L, CB_UV, bpw // CB_UV, rb0, rb1,
                      ov, sem0, sem1, 0)
    _gather_pool_pass(v_hbm, inb, 0, K, CB_N, bpw // CB_N, rb0, rb1,
                      on, sem0, sem1, 0)

    base = wid * bpw
    pltpu.sync_copy(ou, su_hbm.at[pl.ds(base, bpw)])
    pltpu.sync_copy(ov, sv_hbm.at[pl.ds(base, bpw)])
    pltpu.sync_copy(on, sn_hbm.at[pl.ds(base, bpw)])


VB = 32768


def _repack_body(xt_ref, eye_ref, o_ref):
    x = xt_ref[...]
    y = jax.lax.dot_general(x, eye_ref[...], (((0,), (0,)), ((), ())),
                            preferred_element_type=jnp.float32)
    h = y.shape[0] // 2
    o_ref[:, 0:64] = y[0:h, :]
    o_ref[:, 64:128] = y[h:, :]


def _repack(t, D):
    V = t.shape[0]
    grid = (V + VB - 1) // VB
    eye = jnp.eye(D, dtype=jnp.float32)
    return pl.pallas_call(
        _repack_body,
        grid=(grid,),
        in_specs=[
            pl.BlockSpec((D, VB), lambda i: (0, i)),
            pl.BlockSpec((D, D), lambda i: (0, 0)),
        ],
        out_specs=pl.BlockSpec((VB // 2, 2 * D), lambda i: (i, 0)),
        out_shape=jax.ShapeDtypeStruct((grid * (VB // 2), 2 * D), jnp.float32),
    )(t.T, eye)


def _remap_body(up_ref, vp_ref, vn_ref, ou_ref, ov_ref, on_ref):
    sh = VB.bit_length() - 1

    def remap(v):
        return ((v >> sh) << sh) | ((v & (VB // 2 - 1)) << 1) | (
            (v >> (sh - 1)) & 1)
    ou_ref[...] = remap(up_ref[...])
    ov_ref[...] = remap(vp_ref[...])
    on_ref[...] = remap(vn_ref[...])


def _remap_indices(up, vp, vn):
    n_p, n_n = up.shape[0], vn.shape[0]
    ou, ov, on = pl.pallas_call(
        _remap_body,
        out_shape=[jax.ShapeDtypeStruct((n_p // 128, 128), jnp.int32)] * 2
        + [jax.ShapeDtypeStruct((n_n // 128, 128), jnp.int32)],
    )(up.reshape(n_p // 128, 128), vp.reshape(n_p // 128, 128),
      vn.reshape(n_n // 128, 128))
    return ou.reshape(-1), ov.reshape(-1), on.reshape(-1)


def _tc_body(scale, su_ref, sv_ref, sn_ref, o_ref):
    su = su_ref[...]
    sv = sv_ref[...]
    sn = sn_ref[...]
    s = jnp.sum(su * sv, axis=1) * scale
    ns = jnp.sum(su * sn, axis=1) * scale

    def logsig(x):
        return jnp.minimum(x, 0.0) - jnp.log1p(jnp.exp(-jnp.abs(x)))

    loss = logsig(s) + logsig(-ns)
    o_ref[...] = jnp.reshape(-jnp.sum(loss), (1, 1))


def kernel(u_emb, v_emb, u_pos, v_pos, v_neg, batch_size):
    B, L = u_pos.shape
    NNEG = v_neg.shape[2]
    D = u_emb.shape[1]
    K = L * NNEG
    bpw = B // NW

    up = u_pos.astype(jnp.int32).reshape(-1)
    vp = v_pos.astype(jnp.int32).reshape(-1)
    vn = v_neg.astype(jnp.int32).reshape(-1)
    up, vp, vn = _remap_indices(up, vp, vn)

    u_tbl = _repack(u_emb, D)
    v_tbl = _repack(v_emb, D)
    u_tbl = u_tbl.reshape(2 * u_tbl.shape[0], D)
    v_tbl = v_tbl.reshape(2 * v_tbl.shape[0], D)

    sc = functools.partial(
        pl.kernel,
        mesh=plsc.VectorSubcoreMesh(core_axis_name="c", subcore_axis_name="s"),
        out_type=[jax.ShapeDtypeStruct((B, D), jnp.float32)] * 3,
        scratch_types=[
            pltpu.VMEM((bpw * L,), jnp.int32),
            pltpu.VMEM((bpw * L,), jnp.int32),
            pltpu.VMEM((bpw * K,), jnp.int32),
            pltpu.VMEM((400, D), jnp.float32),
            pltpu.VMEM((400, D), jnp.float32),
            pltpu.VMEM((bpw, D), jnp.float32),
            pltpu.VMEM((bpw, D), jnp.float32),
            pltpu.VMEM((bpw, D), jnp.float32),
            pltpu.SemaphoreType.DMA,
            pltpu.SemaphoreType.DMA,
            pltpu.SemaphoreType.DMA,
        ],
        compiler_params=pltpu.CompilerParams(use_tc_tiling_on_sc=False),
    )(functools.partial(_sc_body, B, L, K, D))
    su, sv, sn = sc(u_tbl, v_tbl, up, vp, vn)

    out = pl.pallas_call(
        functools.partial(_tc_body, 1.0 / float(L * L)),
        out_shape=jax.ShapeDtypeStruct((1, 1), jnp.float32),
    )(su, sv, sn)
    return out[0, 0] / jnp.asarray(batch_size, jnp.float32)


# speedup vs baseline: 2.2752x; 1.1399x over previous
import functools

import jax
import jax.numpy as jnp
from jax import lax
from jax.experimental import pallas as pl
from jax.experimental.pallas import tpu as pltpu
from jax.experimental.pallas import tpu_sc as plsc

NC = 2
NS = 16
NW = NC * NS


def _gather_pool_pass(tbl, idx_ref, idx_base, G, CBs, nch, rb0, rb1,
                      out_ref, sem0, sem1, out_base):
    nrows = CBs * G

    def fire(c, rb, sem):
        off = idx_base + c * nrows
        pltpu.async_copy(tbl.at[idx_ref.at[pl.ds(off, nrows)]],
                         rb.at[pl.ds(0, nrows)], sem)

    def drain(rb, sem):
        pltpu.make_async_copy(tbl.at[pl.ds(0, nrows)],
                              rb.at[pl.ds(0, nrows)], sem).wait()

    def compute(c, rb):
        def b_body(bl, _):
            row = out_base + c * CBs + bl
            base = bl * G
            accs = [[rb[base + p, pl.ds(dc * 16, 16)] for p in range(2)]
                    for dc in range(4)]
            for k in range(2, G):
                p = k % 2
                for dc in range(4):
                    accs[dc][p] = accs[dc][p] + rb[base + k, pl.ds(dc * 16, 16)]
            for dc in range(4):
                out_ref[row, pl.ds(dc * 16, 16)] = accs[dc][0] + accs[dc][1]
            return _
        lax.fori_loop(0, CBs, b_body, None)

    fire(0, rb0, sem0)

    def body2(j, _):
        c0 = 2 * j
        drain(rb0, sem0)
        fire(c0 + 1, rb1, sem1)
        compute(c0, rb0)
        drain(rb1, sem1)
        fire(jnp.minimum(c0 + 2, nch - 1), rb0, sem0)
        compute(c0 + 1, rb1)
        return _
    lax.fori_loop(0, nch // 2, body2, None)
    drain(rb0, sem0)


def _sc_body_vn(B, L, K, D, v_hbm, vp_hbm, vn_hbm,
                sv_hbm, sn_hbm, iv, inb, rb0, rb1,
                ov, on, sem0, sem1, semi):
    bpw = B // NW
    wid = lax.axis_index("s") * NC + lax.axis_index("c")

    pltpu.async_copy(vp_hbm.at[pl.ds(wid * (bpw * L), bpw * L)], iv, semi)
    pltpu.async_copy(vn_hbm.at[pl.ds(wid * (bpw * K), bpw * K)], inb, semi)
    pltpu.make_async_copy(vp_hbm.at[pl.ds(0, bpw * L)], iv, semi).wait()
    pltpu.make_async_copy(vn_hbm.at[pl.ds(0, bpw * K)], inb, semi).wait()

    CB_UV = 16
    CB_N = 4
    _gather_pool_pass(v_hbm, iv, 0, L, CB_UV, bpw // CB_UV, rb0, rb1,
                      ov, sem0, sem1, 0)
    _gather_pool_pass(v_hbm, inb, 0, K, CB_N, bpw // CB_N, rb0, rb1,
                      on, sem0, sem1, 0)

    base = wid * bpw
    pltpu.sync_copy(ov, sv_hbm.at[pl.ds(base, bpw)])
    pltpu.sync_copy(on, sn_hbm.at[pl.ds(base, bpw)])


def _sc_body_u(B, L, D, u_hbm, up_hbm, su_hbm, iu, rb0, rb1,
               ou, sem0, sem1, semi):
    bpw = B // NW
    wid = lax.axis_index("s") * NC + lax.axis_index("c")

    pltpu.async_copy(up_hbm.at[pl.ds(wid * (bpw * L), bpw * L)], iu, semi)
    pltpu.make_async_copy(up_hbm.at[pl.ds(0, bpw * L)], iu, semi).wait()

    CB_UV = 16
    _gather_pool_pass(u_hbm, iu, 0, L, CB_UV, bpw // CB_UV, rb0, rb1,
                      ou, sem0, sem1, 0)

    pltpu.sync_copy(ou, su_hbm.at[pl.ds(wid * bpw, bpw)])


VB = 32768


def _repack_body(xt_ref, eye_ref, o_ref):
    x = xt_ref[...]
    y = jax.lax.dot_general(x, eye_ref[...], (((0,), (0,)), ((), ())),
                            preferred_element_type=jnp.float32)
    h = y.shape[0] // 2
    o_ref[:, 0:64] = y[0:h, :]
    o_ref[:, 64:128] = y[h:, :]


def _repack(t, D):
    V = t.shape[0]
    grid = (V + VB - 1) // VB
    eye = jnp.eye(D, dtype=jnp.float32)
    return pl.pallas_call(
        _repack_body,
        grid=(grid,),
        in_specs=[
            pl.BlockSpec((D, VB), lambda i: (0, i)),
            pl.BlockSpec((D, D), lambda i: (0, 0)),
        ],
        out_specs=pl.BlockSpec((VB // 2, 2 * D), lambda i: (i, 0)),
        out_shape=jax.ShapeDtypeStruct((grid * (VB // 2), 2 * D), jnp.float32),
    )(t.T, eye)


def _remap_body(up_ref, vp_ref, vn_ref, ou_ref, ov_ref, on_ref):
    sh = VB.bit_length() - 1

    def remap(v):
        return ((v >> sh) << sh) | ((v & (VB // 2 - 1)) << 1) | (
            (v >> (sh - 1)) & 1)
    ou_ref[...] = remap(up_ref[...])
    ov_ref[...] = remap(vp_ref[...])
    on_ref[...] = remap(vn_ref[...])


def _remap_indices(up, vp, vn):
    n_p, n_n = up.shape[0], vn.shape[0]
    ou, ov, on = pl.pallas_call(
        _remap_body,
        out_shape=[jax.ShapeDtypeStruct((n_p // 128, 128), jnp.int32)] * 2
        + [jax.ShapeDtypeStruct((n_n // 128, 128), jnp.int32)],
    )(up.reshape(n_p // 128, 128), vp.reshape(n_p // 128, 128),
      vn.reshape(n_n // 128, 128))
    return ou.reshape(-1), ov.reshape(-1), on.reshape(-1)


def _tc_body(scale, su_ref, sv_ref, sn_ref, o_ref):
    su = su_ref[...]
    sv = sv_ref[...]
    sn = sn_ref[...]
    s = jnp.sum(su * sv, axis=1) * scale
    ns = jnp.sum(su * sn, axis=1) * scale

    def logsig(x):
        return jnp.minimum(x, 0.0) - jnp.log1p(jnp.exp(-jnp.abs(x)))

    loss = logsig(s) + logsig(-ns)
    o_ref[...] = jnp.reshape(-jnp.sum(loss), (1, 1))


def kernel(u_emb, v_emb, u_pos, v_pos, v_neg, batch_size):
    B, L = u_pos.shape
    NNEG = v_neg.shape[2]
    D = u_emb.shape[1]
    K = L * NNEG
    bpw = B // NW

    up = u_pos.astype(jnp.int32).reshape(-1)
    vp = v_pos.astype(jnp.int32).reshape(-1)
    vn = v_neg.astype(jnp.int32).reshape(-1)
    up, vp, vn = _remap_indices(up, vp, vn)

    v_tbl = _repack(v_emb, D)
    v_tbl = v_tbl.reshape(2 * v_tbl.shape[0], D)

    mesh = plsc.VectorSubcoreMesh(core_axis_name="c", subcore_axis_name="s")
    sc_vn = pl.kernel(
        functools.partial(_sc_body_vn, B, L, K, D),
        mesh=mesh,
        out_type=[jax.ShapeDtypeStruct((B, D), jnp.float32)] * 2,
        scratch_types=[
            pltpu.VMEM((bpw * L,), jnp.int32),
            pltpu.VMEM((bpw * K,), jnp.int32),
            pltpu.VMEM((400, D), jnp.float32),
            pltpu.VMEM((400, D), jnp.float32),
            pltpu.VMEM((bpw, D), jnp.float32),
            pltpu.VMEM((bpw, D), jnp.float32),
            pltpu.SemaphoreType.DMA,
            pltpu.SemaphoreType.DMA,
            pltpu.SemaphoreType.DMA,
        ],
        compiler_params=pltpu.CompilerParams(use_tc_tiling_on_sc=False),
    )
    sv, sn = sc_vn(v_tbl, vp, vn)

    u_tbl = _repack(u_emb, D)
    u_tbl = u_tbl.reshape(2 * u_tbl.shape[0], D)
    sc_u = pl.kernel(
        functools.partial(_sc_body_u, B, L, D),
        mesh=mesh,
        out_type=[jax.ShapeDtypeStruct((B, D), jnp.float32)],
        scratch_types=[
            pltpu.VMEM((bpw * L,), jnp.int32),
            pltpu.VMEM((400, D), jnp.float32),
            pltpu.VMEM((400, D), jnp.float32),
            pltpu.VMEM((bpw, D), jnp.float32),
            pltpu.SemaphoreType.DMA,
            pltpu.SemaphoreType.DMA,
            pltpu.SemaphoreType.DMA,
        ],
        compiler_params=pltpu.CompilerParams(use_tc_tiling_on_sc=False),
    )
    (su,) = sc_u(u_tbl, up)

    out = pl.pallas_call(
        functools.partial(_tc_body, 1.0 / float(L * L)),
        out_shape=jax.ShapeDtypeStruct((1, 1), jnp.float32),
    )(su, sv, sn)
    return out[0, 0] / jnp.asarray(batch_size, jnp.float32)
